# baseline (speedup 1.0000x reference)
import functools

import jax
import jax.numpy as jnp
from jax import lax
from jax.experimental import pallas as pl
from jax.experimental.pallas import tpu as pltpu
from jax.experimental.pallas import tpu_sc as plsc

N, C, D, H, W = 4, 3, 16, 256, 256
HW = H * W
DHW = D * HW
V = N * DHW
PAD = W + 1
R = V + W + 1
NC, NS = 2, 16
NW = NC * NS
PER_W = V // NW
BV = 1024
NGRP = BV // 16
NBLK = PER_W // BV
CHUNK = 128
NCH = 2 * BV // CHUNK

_CP = pltpu.CompilerParams(needs_layout_passes=False, use_tc_tiling_on_sc=False)


def _sc_body(tbl, wfx, out, xb0, xb1, id0, id1, wg0, wg1, va0, va1, ob0, ob1,
             sem_in, sem_g, sem_out):
    cid = lax.axis_index("c")
    sid = lax.axis_index("s")
    wid = sid * NC + cid
    n = wid // (NW // N)
    vloc0 = (wid % (NW // N)) * PER_W
    wf_b = n * (C * DHW)
    n_off = n * DHW + PAD

    xbs, ids, wgs, vas, obs = (xb0, xb1), (id0, id1), (wg0, wg1), \
        (va0, va1), (ob0, ob1)

    def in_start(k, s):
        vloc = vloc0 + jnp.minimum(k, NBLK - 1) * BV
        for c3 in range(C):
            pltpu.async_copy(wfx.at[pl.ds(wf_b + c3 * DHW + vloc, BV)],
                             xbs[s].at[c3], sem_in)

    def in_wait(s):
        for c3 in range(C):
            pltpu.make_async_copy(wfx.at[pl.ds(wf_b, BV)],
                                  xbs[s].at[c3], sem_in).wait()

    def gen(k, s):
        xb, idv, wgv = xbs[s], ids[s], wgs[s]

        def body(i, c2):
            zero = jnp.zeros((16,), jnp.float32)
            ione = jnp.ones((16,), jnp.int32)
            izero = jnp.zeros((16,), jnp.int32)
            sl = pl.ds(i * 16, 16)
            x = xb[0, sl]
            y = xb[1, sl]
            z = xb[2, sl]
            ix = ((x + 1.0) * jnp.float32(W) - 1.0) * 0.5
            iy = ((y + 1.0) * jnp.float32(H) - 1.0) * 0.5
            iz = ((z + 1.0) * jnp.float32(D) - 1.0) * 0.5

            def fl(v):
                t = v.astype(jnp.int32)
                return t - jnp.where(t.astype(jnp.float32) > v, ione, izero)

            x0 = fl(ix)
            y0 = fl(iy)
            z0 = fl(iz)
            wx1 = ix - x0.astype(jnp.float32)
            wx0 = 1.0 - wx1
            wy1 = iy - y0.astype(jnp.float32)
            wy0 = 1.0 - wy1
            wz1 = iz - z0.astype(jnp.float32)
            wz0 = 1.0 - wz1
            wx0 = jnp.where((x0 >= 0) & (x0 <= W - 1), wx0, zero)
            wx1 = jnp.where((x0 >= -1) & (x0 <= W - 2), wx1, zero)
            wy0 = jnp.where((y0 >= 0) & (y0 <= H - 1), wy0, zero)
            wy1 = jnp.where((y0 >= -1) & (y0 <= H - 2), wy1, zero)
            wz0 = jnp.where((z0 >= 0) & (z0 <= D - 1), wz0, zero)
            wz1 = jnp.where((z0 >= -1) & (z0 <= D - 2), wz1, zero)
            z0c = jnp.clip(z0, 0, D - 1)
            z1c = jnp.clip(z0 + 1, 0, D - 1)
            yb = y0 * W + x0 + n_off
            idv[sl] = jnp.clip(yb + z0c * HW, 0, R - 1)
            idv[pl.ds(BV + i * 16, 16)] = jnp.clip(yb + z1c * HW, 0, R - 1)
            w00 = wy0 * wx0
            w01 = wy0 * wx1
            w10 = wy1 * wx0
            w11 = wy1 * wx1
            wgv[0, sl] = wz0 * w00
            wgv[1, sl] = wz0 * w01
            wgv[2, sl] = wz0 * w10
            wgv[3, sl] = wz0 * w11
            wgv[4, sl] = wz1 * w00
            wgv[5, sl] = wz1 * w01
            wgv[6, sl] = wz1 * w10
            wgv[7, sl] = wz1 * w11
            return c2

        lax.fori_loop(0, NGRP, body, 0)

    def gather_start(s):
        for kc in range(NCH):
            pltpu.async_copy(tbl.at[ids[s].at[pl.ds(kc * CHUNK, CHUNK)]],
                             vas[s].at[pl.ds(kc * CHUNK, CHUNK)], sem_g)

    def gather_wait(s):
        for kc in range(NCH):
            pltpu.make_async_copy(
                tbl.at[ids[s].at[pl.ds(kc * CHUNK, CHUNK)]],
                vas[s].at[pl.ds(kc * CHUNK, CHUNK)], sem_g).wait()

    def comb(k, s):
        valv, wgv, obv = vas[s], wgs[s], obs[s]

        def body(i, c2):
            iota = lax.iota(jnp.int32, 16)
            zero = jnp.zeros((16,), jnp.float32)
            sl = pl.ds(i * 16, 16)
            rows0 = iota + i * 16
            rows1 = rows0 + BV
            acc = [zero, zero, zero]
            for g, rows in ((0, rows0), (1, rows1)):
                for slot in range(4):
                    wq = wgv[g * 4 + slot, sl]
                    for ch in range(3):
                        col = jnp.full((16,), slot * 4 + ch, jnp.int32)
                        vv = plsc.load_gather(valv, [rows, col])
                        acc[ch] = acc[ch] + wq * vv
            obv[0, sl] = acc[0]
            obv[1, sl] = acc[1]
            obv[2, sl] = acc[2]
            return c2

        lax.fori_loop(0, NGRP, body, 0)

    def out_start(k, s):
        vloc = vloc0 + k * BV
        for c3 in range(C):
            pltpu.async_copy(obs[s].at[c3],
                             out.at[pl.ds(wf_b + c3 * DHW + vloc, BV)],
                             sem_out)

    def out_wait(s):
        for c3 in range(C):
            pltpu.make_async_copy(obs[s].at[c3],
                                  out.at[pl.ds(wf_b, BV)], sem_out).wait()

    def stage(k, cur, nxt):
        in_wait(nxt)
        in_start(k + 2, cur)
        gen(jnp.minimum(k + 1, NBLK - 1), nxt)
        gather_start(nxt)
        gather_wait(cur)

        @pl.when(k >= 2)
        def _():
            out_wait(cur)

        comb(k, cur)
        out_start(k, cur)

    in_start(0, 0)
    in_wait(0)
    gen(0, 0)
    gather_start(0)
    in_start(1, 1)

    def pair(m, carry):
        stage(2 * m, 0, 1)
        stage(2 * m + 1, 1, 0)
        return carry

    lax.fori_loop(0, NBLK // 2, pair, 0)
    gather_wait(0)
    in_wait(1)
    out_wait(0)
    out_wait(1)


@functools.cache
def _grid_sample_sc():
    return pl.kernel(
        _sc_body,
        out_type=jax.ShapeDtypeStruct((N * C * DHW,), jnp.float32),
        mesh=plsc.VectorSubcoreMesh(core_axis_name="c", subcore_axis_name="s"),
        compiler_params=_CP,
        scratch_types=[
            pltpu.VMEM((C, BV), jnp.float32),
            pltpu.VMEM((C, BV), jnp.float32),
            pltpu.VMEM((2 * BV,), jnp.int32),
            pltpu.VMEM((2 * BV,), jnp.int32),
            pltpu.VMEM((8, BV), jnp.float32),
            pltpu.VMEM((8, BV), jnp.float32),
            pltpu.VMEM((2 * BV, 16), jnp.float32),
            pltpu.VMEM((2 * BV, 16), jnp.float32),
            pltpu.VMEM((C, BV), jnp.float32),
            pltpu.VMEM((C, BV), jnp.float32),
            pltpu.SemaphoreType.DMA,
            pltpu.SemaphoreType.DMA,
            pltpu.SemaphoreType.DMA,
        ],
    )


BT = 2048
NBLK2 = PER_W // BT
SZ = BT + 264
MB0 = 264
TOT = MB0 + SZ + 264
_OFFS = (0, 1, W, W + 1)


def _build_body(fr, tbl, s00, s01, s02, s10, s11, s12, ob0, ob1,
                sem_in, sem_out):
    cid = lax.axis_index("c")
    sid = lax.axis_index("s")
    wid = sid * NC + cid
    n = wid // (NW // N)
    j = wid % (NW // N)
    row0 = n * DHW + j * PER_W
    bufsets = ((s00, s01, s02), (s10, s11, s12))
    obs = (ob0, ob1)

    def zinit(i, c2):
        zero = jnp.zeros((16,), jnp.float32)
        iota = lax.iota(jnp.int32, 16)
        for bs in bufsets:
            for b in bs:
                plsc.store_scatter(b, [iota + i * 16], zero)
                plsc.store_scatter(b, [iota + (TOT - 272) + i * 16], zero)
        return c2

    lax.fori_loop(0, 17, zinit, 0)

    @pl.when((j == 0) & (n > 0))
    def _prefix():
        for c3 in range(C):
            src = (n - 1) * (C * DHW) + c3 * DHW + (DHW - MB0)
            pltpu.sync_copy(fr.at[pl.ds(src, MB0)],
                            bufsets[0][c3].at[pl.ds(0, MB0)])

    def src_align(r0):
        base = r0 - 257 - n * DHW
        srcA = pl.multiple_of(jnp.clip(base & ~7, 0, DHW - SZ), 8)
        return srcA, base - srcA

    def in_start(r0, s):
        srcA, _ = src_align(r0)
        for c3 in range(C):
            src = n * (C * DHW) + c3 * DHW + srcA
            pltpu.async_copy(fr.at[pl.ds(src, SZ)],
                             bufsets[s][c3].at[pl.ds(MB0, SZ)], sem_in)

    def in_wait(s):
        for c3 in range(C):
            pltpu.make_async_copy(fr.at[pl.ds(0, SZ)],
                                  bufsets[s][c3].at[pl.ds(MB0, SZ)],
                                  sem_in).wait()

    def asm(r0, s):
        _, delta = src_align(r0)
        bufs, obuf = bufsets[s], obs[s]

        def body(g, c2):
            iota = lax.iota(jnp.int32, 16)
            rowv = iota + g * 16
            for si, off_s in enumerate(_OFFS):
                lidx = iota + (MB0 + g * 16 + off_s + delta)
                for c3 in range(C):
                    vals = plsc.load_gather(bufs[c3], [lidx])
                    colv = jnp.full((16,), si * 4 + c3, jnp.int32)
                    plsc.store_scatter(obuf, [rowv, colv], vals)
            return c2

        lax.fori_loop(0, BT // 16, body, 0)

    def out_start(r0, s):
        pltpu.async_copy(obs[s], tbl.at[pl.ds(r0, BT)], sem_out)

    def out_wait(s):
        pltpu.make_async_copy(obs[s], tbl.at[pl.ds(0, BT)], sem_out).wait()

    def stage(k, cur):
        r0 = row0 + k * BT
        in_start(row0 + jnp.minimum(k + 1, NBLK2 - 1) * BT, 1 - cur)
        in_wait(cur)
        asm(r0, cur)

        @pl.when(k >= 2)
        def _():
            out_wait(cur)

        out_start(r0, cur)

    in_start(row0, 0)

    def pair(m, carry):
        stage(2 * m, 0)
        stage(2 * m + 1, 1)
        return carry

    lax.fori_loop(0, NBLK2 // 2, pair, 0)
    in_wait(1)
    out_wait(0)
    out_wait(1)

    @pl.when(wid == NW - 1)
    def _tail():
        r0 = R - BT
        in_start(r0, 0)
        in_wait(0)
        asm(r0, 0)
        pltpu.sync_copy(obs[0], tbl.at[pl.ds(r0, BT)])


@functools.cache
def _build_sc():
    return pl.kernel(
        _build_body,
        out_type=jax.ShapeDtypeStruct((R, 16), jnp.float32),
        mesh=plsc.VectorSubcoreMesh(core_axis_name="c", subcore_axis_name="s"),
        compiler_params=_CP,
        scratch_types=[
            pltpu.VMEM((TOT,), jnp.float32),
            pltpu.VMEM((TOT,), jnp.float32),
            pltpu.VMEM((TOT,), jnp.float32),
            pltpu.VMEM((TOT,), jnp.float32),
            pltpu.VMEM((TOT,), jnp.float32),
            pltpu.VMEM((TOT,), jnp.float32),
            pltpu.VMEM((BT, 16), jnp.float32),
            pltpu.VMEM((BT, 16), jnp.float32),
            pltpu.SemaphoreType.DMA,
            pltpu.SemaphoreType.DMA,
        ],
    )


def kernel(frames, wf):
    tbl = _build_sc()(frames.reshape(-1))
    out = _grid_sample_sc()(tbl, wf.reshape(-1))
    return out.reshape(N, C, D, H, W)


# speedup vs baseline: 1.0039x; 1.0039x over previous
import functools

import jax
import jax.numpy as jnp
from jax import lax
from jax.experimental import pallas as pl
from jax.experimental.pallas import tpu as pltpu
from jax.experimental.pallas import tpu_sc as plsc

N, C, D, H, W = 4, 3, 16, 256, 256
HW = H * W
DHW = D * HW
V = N * DHW
PAD = W + 1
R = V + W + 1
NC, NS = 2, 16
NW = NC * NS
PER_W = V // NW
BV = 1024
NGRP = BV // 16
NBLK = PER_W // BV
CHUNK = 128
NCH = 2 * BV // CHUNK

_CP = pltpu.CompilerParams(needs_layout_passes=False, use_tc_tiling_on_sc=False)


def _sc_body(tbl, wfx, out, xb0, xb1, id0, id1, wg0, wg1, va0, va1, ob0, ob1,
             sem_in, sem_g, sem_out):
    cid = lax.axis_index("c")
    sid = lax.axis_index("s")
    wid = sid * NC + cid
    n = wid // (NW // N)
    vloc0 = (wid % (NW // N)) * PER_W
    wf_b = n * (C * DHW)
    n_off = n * DHW + PAD

    xbs, ids, wgs, vas, obs = (xb0, xb1), (id0, id1), (wg0, wg1), \
        (va0, va1), (ob0, ob1)

    def in_start(k, s):
        vloc = vloc0 + jnp.minimum(k, NBLK - 1) * BV
        for c3 in range(C):
            pltpu.async_copy(wfx.at[pl.ds(wf_b + c3 * DHW + vloc, BV)],
                             xbs[s].at[c3], sem_in)

    def in_wait(s):
        for c3 in range(C):
            pltpu.make_async_copy(wfx.at[pl.ds(wf_b, BV)],
                                  xbs[s].at[c3], sem_in).wait()

    def gen(k, s):
        xb, idv, wgv = xbs[s], ids[s], wgs[s]

        def body(i, c2):
            zero = jnp.zeros((16,), jnp.float32)
            ione = jnp.ones((16,), jnp.int32)
            izero = jnp.zeros((16,), jnp.int32)
            sl = pl.ds(i * 16, 16)
            x = xb[0, sl]
            y = xb[1, sl]
            z = xb[2, sl]
            ix = ((x + 1.0) * jnp.float32(W) - 1.0) * 0.5
            iy = ((y + 1.0) * jnp.float32(H) - 1.0) * 0.5
            iz = ((z + 1.0) * jnp.float32(D) - 1.0) * 0.5

            def fl(v):
                t = v.astype(jnp.int32)
                return t - jnp.where(t.astype(jnp.float32) > v, ione, izero)

            x0 = fl(ix)
            y0 = fl(iy)
            z0 = fl(iz)
            wx1 = ix - x0.astype(jnp.float32)
            wx0 = 1.0 - wx1
            wy1 = iy - y0.astype(jnp.float32)
            wy0 = 1.0 - wy1
            wz1 = iz - z0.astype(jnp.float32)
            wz0 = 1.0 - wz1
            wx0 = jnp.where((x0 >= 0) & (x0 <= W - 1), wx0, zero)
            wx1 = jnp.where((x0 >= -1) & (x0 <= W - 2), wx1, zero)
            wy0 = jnp.where((y0 >= 0) & (y0 <= H - 1), wy0, zero)
            wy1 = jnp.where((y0 >= -1) & (y0 <= H - 2), wy1, zero)
            wz0 = jnp.where((z0 >= 0) & (z0 <= D - 1), wz0, zero)
            wz1 = jnp.where((z0 >= -1) & (z0 <= D - 2), wz1, zero)
            z0c = jnp.clip(z0, 0, D - 1)
            z1c = jnp.clip(z0 + 1, 0, D - 1)
            yb = y0 * W + x0 + n_off
            idv[sl] = jnp.clip(yb + z0c * HW, 0, R - 1)
            idv[pl.ds(BV + i * 16, 16)] = jnp.clip(yb + z1c * HW, 0, R - 1)
            w00 = wy0 * wx0
            w01 = wy0 * wx1
            w10 = wy1 * wx0
            w11 = wy1 * wx1
            wgv[0, sl] = wz0 * w00
            wgv[1, sl] = wz0 * w01
            wgv[2, sl] = wz0 * w10
            wgv[3, sl] = wz0 * w11
            wgv[4, sl] = wz1 * w00
            wgv[5, sl] = wz1 * w01
            wgv[6, sl] = wz1 * w10
            wgv[7, sl] = wz1 * w11
            return c2

        lax.fori_loop(0, NGRP, body, 0)

    def gather_start(s):
        for kc in range(NCH):
            pltpu.async_copy(tbl.at[ids[s].at[pl.ds(kc * CHUNK, CHUNK)]],
                             vas[s].at[pl.ds(kc * CHUNK, CHUNK)], sem_g)

    def gather_wait(s):
        for kc in range(NCH):
            pltpu.make_async_copy(
                tbl.at[ids[s].at[pl.ds(kc * CHUNK, CHUNK)]],
                vas[s].at[pl.ds(kc * CHUNK, CHUNK)], sem_g).wait()

    def comb(k, s):
        valv, wgv, obv = vas[s], wgs[s], obs[s]

        def body(i, c2):
            iota = lax.iota(jnp.int32, 16)
            zero = jnp.zeros((16,), jnp.float32)
            sl = pl.ds(i * 16, 16)
            rows0 = iota + i * 16
            rows1 = rows0 + BV
            acc = [zero, zero, zero]
            for g, rows in ((0, rows0), (1, rows1)):
                for slot in range(4):
                    wq = wgv[g * 4 + slot, sl]
                    for ch in range(3):
                        col = jnp.full((16,), slot * 4 + ch, jnp.int32)
                        vv = plsc.load_gather(valv, [rows, col])
                        acc[ch] = acc[ch] + wq * vv
            obv[0, sl] = acc[0]
            obv[1, sl] = acc[1]
            obv[2, sl] = acc[2]
            return c2

        lax.fori_loop(0, NGRP, body, 0)

    def out_start(k, s):
        vloc = vloc0 + k * BV
        for c3 in range(C):
            pltpu.async_copy(obs[s].at[c3],
                             out.at[pl.ds(wf_b + c3 * DHW + vloc, BV)],
                             sem_out)

    def out_wait(s):
        for c3 in range(C):
            pltpu.make_async_copy(obs[s].at[c3],
                                  out.at[pl.ds(wf_b, BV)], sem_out).wait()

    def stage(k, cur, nxt):
        in_wait(nxt)
        in_start(k + 2, cur)
        gen(jnp.minimum(k + 1, NBLK - 1), nxt)
        gather_start(nxt)
        gather_wait(cur)

        @pl.when(k >= 2)
        def _():
            out_wait(cur)

        comb(k, cur)
        out_start(k, cur)

    in_start(0, 0)
    in_wait(0)
    gen(0, 0)
    gather_start(0)
    in_start(1, 1)

    def pair(m, carry):
        stage(2 * m, 0, 1)
        stage(2 * m + 1, 1, 0)
        return carry

    lax.fori_loop(0, NBLK // 2, pair, 0)
    gather_wait(0)
    in_wait(1)
    out_wait(0)
    out_wait(1)


@functools.cache
def _grid_sample_sc():
    return pl.kernel(
        _sc_body,
        out_type=jax.ShapeDtypeStruct((N * C * DHW,), jnp.float32),
        mesh=plsc.VectorSubcoreMesh(core_axis_name="c", subcore_axis_name="s"),
        compiler_params=_CP,
        scratch_types=[
            pltpu.VMEM((C, BV), jnp.float32),
            pltpu.VMEM((C, BV), jnp.float32),
            pltpu.VMEM((2 * BV,), jnp.int32),
            pltpu.VMEM((2 * BV,), jnp.int32),
            pltpu.VMEM((8, BV), jnp.float32),
            pltpu.VMEM((8, BV), jnp.float32),
            pltpu.VMEM((2 * BV, 16), jnp.float32),
            pltpu.VMEM((2 * BV, 16), jnp.float32),
            pltpu.VMEM((C, BV), jnp.float32),
            pltpu.VMEM((C, BV), jnp.float32),
            pltpu.SemaphoreType.DMA,
            pltpu.SemaphoreType.DMA,
            pltpu.SemaphoreType.DMA,
        ],
    )


BT = 1024
NBLK2 = PER_W // BT
SZ = BT + 264
MB0 = 264
TOT = MB0 + SZ + 264
_OFFS = (0, 1, W, W + 1)


def _build_body(fr, tbl, s00, s01, s02, s10, s11, s12, ob0, ob1,
                sem_in, sem_out):
    cid = lax.axis_index("c")
    sid = lax.axis_index("s")
    wid = sid * NC + cid
    n = wid // (NW // N)
    j = wid % (NW // N)
    row0 = n * DHW + j * PER_W
    bufsets = ((s00, s01, s02), (s10, s11, s12))
    obs = (ob0, ob1)

    def zinit(i, c2):
        zero = jnp.zeros((16,), jnp.float32)
        iota = lax.iota(jnp.int32, 16)
        for bs in bufsets:
            for b in bs:
                plsc.store_scatter(b, [iota + i * 16], zero)
                plsc.store_scatter(b, [iota + (TOT - 272) + i * 16], zero)
        return c2

    lax.fori_loop(0, 17, zinit, 0)

    @pl.when((j == 0) & (n > 0))
    def _prefix():
        for c3 in range(C):
            src = (n - 1) * (C * DHW) + c3 * DHW + (DHW - MB0)
            pltpu.sync_copy(fr.at[pl.ds(src, MB0)],
                            bufsets[0][c3].at[pl.ds(0, MB0)])

    def src_align(r0):
        base = r0 - 257 - n * DHW
        srcA = pl.multiple_of(jnp.clip(base & ~7, 0, DHW - SZ), 8)
        return srcA, base - srcA

    def in_start(r0, s):
        srcA, _ = src_align(r0)
        for c3 in range(C):
            src = n * (C * DHW) + c3 * DHW + srcA
            pltpu.async_copy(fr.at[pl.ds(src, SZ)],
                             bufsets[s][c3].at[pl.ds(MB0, SZ)], sem_in)

    def in_wait(s):
        for c3 in range(C):
            pltpu.make_async_copy(fr.at[pl.ds(0, SZ)],
                                  bufsets[s][c3].at[pl.ds(MB0, SZ)],
                                  sem_in).wait()

    def asm(r0, s):
        _, delta = src_align(r0)
        bufs, obuf = bufsets[s], obs[s]

        def body(g, c2):
            iota = lax.iota(jnp.int32, 16)
            rowv = iota + g * 16
            for si, off_s in enumerate(_OFFS):
                lidx = iota + (MB0 + g * 16 + off_s + delta)
                for c3 in range(C):
                    vals = plsc.load_gather(bufs[c3], [lidx])
                    colv = jnp.full((16,), si * 4 + c3, jnp.int32)
                    plsc.store_scatter(obuf, [rowv, colv], vals)
            return c2

        lax.fori_loop(0, BT // 16, body, 0)

    def out_start(r0, s):
        pltpu.async_copy(obs[s], tbl.at[pl.ds(r0, BT)], sem_out)

    def out_wait(s):
        pltpu.make_async_copy(obs[s], tbl.at[pl.ds(0, BT)], sem_out).wait()

    def stage(k, cur):
        r0 = row0 + k * BT
        in_start(row0 + jnp.minimum(k + 1, NBLK2 - 1) * BT, 1 - cur)
        in_wait(cur)
        asm(r0, cur)

        @pl.when(k >= 2)
        def _():
            out_wait(cur)

        out_start(r0, cur)

    in_start(row0, 0)

    def pair(m, carry):
        stage(2 * m, 0)
        stage(2 * m + 1, 1)
        return carry

    lax.fori_loop(0, NBLK2 // 2, pair, 0)
    in_wait(1)
    out_wait(0)
    out_wait(1)

    @pl.when(wid == NW - 1)
    def _tail():
        r0 = R - BT
        in_start(r0, 0)
        in_wait(0)
        asm(r0, 0)
        pltpu.sync_copy(obs[0], tbl.at[pl.ds(r0, BT)])


@functools.cache
def _build_sc():
    return pl.kernel(
        _build_body,
        out_type=jax.ShapeDtypeStruct((R, 16), jnp.float32),
        mesh=plsc.VectorSubcoreMesh(core_axis_name="c", subcore_axis_name="s"),
        compiler_params=_CP,
        scratch_types=[
            pltpu.VMEM((TOT,), jnp.float32),
            pltpu.VMEM((TOT,), jnp.float32),
            pltpu.VMEM((TOT,), jnp.float32),
            pltpu.VMEM((TOT,), jnp.float32),
            pltpu.VMEM((TOT,), jnp.float32),
            pltpu.VMEM((TOT,), jnp.float32),
            pltpu.VMEM((BT, 16), jnp.float32),
            pltpu.VMEM((BT, 16), jnp.float32),
            pltpu.SemaphoreType.DMA,
            pltpu.SemaphoreType.DMA,
        ],
    )


def kernel(frames, wf):
    tbl = _build_sc()(frames.reshape(-1))
    out = _grid_sample_sc()(tbl, wf.reshape(-1))
    return out.reshape(N, C, D, H, W)
